# baseline (device time: 13635 ns/iter reference)
import jax
import jax.numpy as jnp
from jax import lax
from jax.experimental import pallas as pl
from jax.experimental.pallas import tpu as pltpu

N_DEV = 32

_ORDER = (
    16, 8, 24, 7, 25, 4, 28, 9, 23, 6, 26, 12, 20, 1, 31, 3,
    29, 5, 15, 17, 27, 11, 21, 13, 19, 10, 22, 2, 30, 14, 18,
)


def kernel(x):
    m_per, n = x.shape

    def body(x_ref, out_ref, comm_ref, send_sems, recv_sems):
        my_pos = lax.axis_index("i")

        barrier_sem = pltpu.get_barrier_semaphore()
        for d in range(1, N_DEV):
            pl.semaphore_signal(
                barrier_sem, inc=1,
                device_id=((my_pos + d) % N_DEV,),
                device_id_type=pl.DeviceIdType.MESH,
            )

        xv = x_ref[:, :]
        row = lax.broadcasted_iota(jnp.int32, xv.shape, 0)
        m0 = jnp.max(xv, axis=0)
        cand = jnp.where(xv == m0[None, :], row, jnp.int32(2**30))
        li = jnp.min(cand, axis=0)
        gi = (li + my_pos * m_per).astype(jnp.float32)
        comm_ref[0, 0, :] = m0
        comm_ref[0, 1, :] = gi

        pl.semaphore_wait(barrier_sem, N_DEV - 1)

        rdmas = []
        for s, d in enumerate(_ORDER, start=1):
            rdma = pltpu.make_async_remote_copy(
                src_ref=comm_ref.at[0],
                dst_ref=comm_ref.at[s],
                send_sem=send_sems.at[s],
                recv_sem=recv_sems.at[s],
                device_id=((my_pos + d) % N_DEV,),
                device_id_type=pl.DeviceIdType.MESH,
            )
            rdma.start()
            rdmas.append(rdma)

        best = None
        best_idx = None
        CHUNK = 8
        for lo in range(0, N_DEV, CHUNK):
            hi = min(lo + CHUNK, N_DEV)
            for d in range(max(lo, 1), hi):
                rdmas[d - 1].wait()
            vals = comm_ref[lo:hi, 0, :]
            idxs = comm_ref[lo:hi, 1, :]
            m = jnp.max(vals, axis=0)
            pick = jnp.where(vals == m[None, :], idxs, jnp.float32(jnp.inf))
            mi = jnp.min(pick, axis=0)
            if best is None:
                best, best_idx = m, mi
            else:
                take = (m > best) | ((m == best) & (mi < best_idx))
                best = jnp.where(take, m, best)
                best_idx = jnp.where(take, mi, best_idx)
        out_ref[0, :] = best
        out_ref[1, :] = best_idx

    return pl.pallas_call(
        body,
        out_shape=jax.ShapeDtypeStruct((2, n), jnp.float32),
        in_specs=[pl.BlockSpec(memory_space=pltpu.VMEM)],
        out_specs=pl.BlockSpec(memory_space=pltpu.VMEM),
        scratch_shapes=[
            pltpu.VMEM((N_DEV, 2, n), jnp.float32),
            pltpu.SemaphoreType.DMA((N_DEV,)),
            pltpu.SemaphoreType.DMA((N_DEV,)),
        ],
        compiler_params=pltpu.CompilerParams(collective_id=0),
    )(x)
